# baseline (speedup 1.0000x reference)
import functools

import jax
import jax.numpy as jnp
from jax import lax
from jax.experimental import pallas as pl
from jax.experimental.pallas import tpu as pltpu
from jax.experimental.pallas import tpu_sc as plsc

_B, _F, _N = 8, 3, 2048
_K = 32
_KS = 16
_L = 16
_NW = 32
_ROWS = (_B * _N) // _NW
_WPB = _NW // _B
_CHUNKS = _N // _L
_CAPV = 16
_CAP = _CAPV * _L
_RCAP = 64
_PRUNE_AT = 48
_GROUP = 16
_NEG = float("-inf")


def _sort_desc(k, v):
    return plsc.sort_key_val(k, v, descending=True)


_GDN = lax.GatherDimensionNumbers(offset_dims=(), collapsed_slice_dims=(0,),
                                  start_index_map=(0,))


def _reg_gather(v, idx):
    return lax.gather(v, idx.reshape(_L, 1), _GDN, (1,),
                      mode=lax.GatherScatterMode.PROMISE_IN_BOUNDS)


def _bf16_round(v):
    u = plsc.bitcast(v, jnp.uint32)
    r = u + jnp.uint32(0x7FFF) + ((u >> jnp.uint32(16)) & jnp.uint32(1))
    r = r & jnp.uint32(0xFFFF0000)
    return plsc.bitcast(r, jnp.float32)


def _rev(a):
    return lax.rev(a, dimensions=(0,))


def _merge16(ak, av, bk, bv):
    brk, brv = _rev(bk), _rev(bv)
    m = ak >= brk
    hk = jnp.where(m, ak, brk)
    hv = jnp.where(m, av, brv)
    lk = jnp.where(m, brk, ak)
    lv = jnp.where(m, brv, av)
    hk, hv = _sort_desc(hk, hv)
    lk, lv = _sort_desc(lk, lv)
    return hk, hv, lk, lv


def _merge32_top32(a0k, a0v, a1k, a1v, b0k, b0v, b1k, b1v):
    rb0k, rb0v = _rev(b0k), _rev(b0v)
    rb1k, rb1v = _rev(b1k), _rev(b1v)
    m0 = a0k >= rb1k
    h0k = jnp.where(m0, a0k, rb1k)
    h0v = jnp.where(m0, a0v, rb1v)
    m1 = a1k >= rb0k
    h1k = jnp.where(m1, a1k, rb0k)
    h1v = jnp.where(m1, a1v, rb0v)
    m = h0k >= h1k
    pk = jnp.where(m, h0k, h1k)
    pv = jnp.where(m, h0v, h1v)
    qk = jnp.where(m, h1k, h0k)
    qv = jnp.where(m, h1v, h0v)
    pk, pv = _sort_desc(pk, pv)
    qk, qv = _sort_desc(qk, qv)
    return pk, pv, qk, qv


def _sc_body(x0_hbm, x1_hbm, x2_hbm, kern_hbm, op0_hbm, sidx_hbm, adj_hbm,
             x0_v, x1_v, x2_v, xb0_v, xb1_v, xb2_v, xx_v, scorebuf,
             wbuf, sidx_stage, adj_stage, kern_v, op0_v):
    cid = lax.axis_index("c")
    sid = lax.axis_index("s")
    w = sid * 2 + cid
    b = w // _WPB
    rbase = (w % _WPB) * _ROWS

    pltpu.sync_copy(x0_hbm.at[pl.ds(b * _N, _N)], x0_v)
    pltpu.sync_copy(x1_hbm.at[pl.ds(b * _N, _N)], x1_v)
    pltpu.sync_copy(x2_hbm.at[pl.ds(b * _N, _N)], x2_v)
    pltpu.sync_copy(kern_hbm, kern_v)
    pltpu.sync_copy(op0_hbm, op0_v)

    def xx_body(i, _):
        a0 = x0_v[pl.ds(i * _L, _L)]
        a1 = x1_v[pl.ds(i * _L, _L)]
        a2 = x2_v[pl.ds(i * _L, _L)]
        xx_v[pl.ds(i * _L, _L)] = a0 * a0 + a1 * a1 + a2 * a2
        xb0_v[pl.ds(i * _L, _L)] = _bf16_round(a0)
        xb1_v[pl.ds(i * _L, _L)] = _bf16_round(a1)
        xb2_v[pl.ds(i * _L, _L)] = _bf16_round(a2)
        return 0
    lax.fori_loop(0, _CHUNKS, xx_body, 0)

    kv0 = _bf16_round(kern_v[pl.ds(0, _L)])
    kv1 = _bf16_round(kern_v[pl.ds(_L, _L)])
    kv2 = _bf16_round(kern_v[pl.ds(2 * _L, _L)])
    opv = op0_v[...]
    iota = lax.iota(jnp.int32, _L)
    ninf = jnp.full((_L,), _NEG, jnp.float32)
    zero_i = jnp.zeros((_L,), jnp.int32)
    boff = b * _N

    def process_row(r, rr):
        row = rbase + r
        qi = jnp.full((_L,), row, jnp.int32)
        q0 = plsc.load_gather(xb0_v, [qi])
        q1 = plsc.load_gather(xb1_v, [qi])
        q2 = plsc.load_gather(xb2_v, [qi])

        def score_chunk(c):
            a0 = xb0_v[pl.ds(c * _L, _L)]
            a1 = xb1_v[pl.ds(c * _L, _L)]
            a2 = xb2_v[pl.ds(c * _L, _L)]
            xxv = xx_v[pl.ds(c * _L, _L)]
            dot = q0 * a0 + q1 * a1 + q2 * a2
            return dot + dot - xxv

        def score_loop(c, _):
            scorebuf[pl.ds(c * _L, _L)] = score_chunk(c)
            return 0
        lax.fori_loop(0, _CHUNKS, score_loop, 0)

        stack = []
        for i in range(_CHUNKS // 2):
            k0 = scorebuf[pl.ds((2 * i) * _L, _L)]
            k1 = scorebuf[pl.ds((2 * i + 1) * _L, _L)]
            sk0, sv0 = _sort_desc(k0, iota + (2 * i) * _L)
            sk1, sv1 = _sort_desc(k1, iota + (2 * i + 1) * _L)
            run = _merge16(sk0, sv0, sk1, sv1)
            lvl = 0
            while stack and stack[-1][0] == lvl:
                prev = stack.pop()[1]
                run = _merge32_top32(*prev, *run)
                lvl += 1
            stack.append((lvl, run))
        run = stack.pop()[1]
        while stack:
            run = _merge32_top32(*stack.pop()[1], *run)
        pk, pv, qk, qv = run
        i0, i1 = pv, qv

        sidx_stage[pl.ds(rr * _K, _L)] = i0 + boff
        sidx_stage[pl.ds(rr * _K + _L, _L)] = i1 + boff

        g0a = plsc.load_gather(x0_v, [i0])
        g1a = plsc.load_gather(x1_v, [i0])
        g2a = plsc.load_gather(x2_v, [i0])
        g0b = plsc.load_gather(x0_v, [i1])
        g1b = plsc.load_gather(x1_v, [i1])
        g2b = plsc.load_gather(x2_v, [i1])
        c0 = _reg_gather(g0a, zero_i)
        c1 = _reg_gather(g1a, zero_i)
        c2 = _reg_gather(g2a, zero_i)
        db0a = _bf16_round(g0a - c0)
        db0b = _bf16_round(g0b - c0)
        db1a = _bf16_round(g1a - c1)
        db1b = _bf16_round(g1b - c1)
        db2a = _bf16_round(g2a - c2)
        db2b = _bf16_round(g2b - c2)

        ssum = jnp.zeros((_L,), jnp.float32)
        for j in range(_K):
            jv = jnp.full((_L,), j % _L, jnp.int32)
            if j < _L:
                d0 = _reg_gather(db0a, jv)
                d1 = _reg_gather(db1a, jv)
                d2 = _reg_gather(db2a, jv)
            else:
                d0 = _reg_gather(db0b, jv)
                d1 = _reg_gather(db1b, jv)
                d2 = _reg_gather(db2b, jv)
            wj = d0 * kv0 + d1 * kv1 + d2 * kv2
            if j == 0:
                wj = wj + opv
            wj = jnp.maximum(wj, 0.0)
            ssum = ssum + wj
            wbuf[pl.ds(j * _L, _L)] = wj

        r1 = 1.0 / (ssum + 1e-6)
        s2 = jnp.zeros((_L,), jnp.float32)
        for j in range(_K):
            a = wbuf[pl.ds(j * _L, _L)] * r1
            a = a * a
            s2 = s2 + a
            wbuf[pl.ds(j * _L, _L)] = a

        r2 = 1.0 / (s2 + 1e-6)
        for j in range(_K):
            v = wbuf[pl.ds(j * _L, _L)] * r2
            v = jnp.where(v > 0.1, v, 0.0)
            adj_stage[pl.ds((rr * _K + j) * _KS, _KS)] = v
        return 0

    def group_body(g, _):
        def row_body(rr, _):
            return process_row(g * _GROUP + rr, rr)
        lax.fori_loop(0, _GROUP, row_body, 0)
        base = w * _ROWS + g * _GROUP
        pltpu.sync_copy(adj_stage,
                        adj_hbm.at[pl.ds(base * _K * _KS, _GROUP * _K * _KS)])
        pltpu.sync_copy(sidx_stage,
                        sidx_hbm.at[pl.ds(base * _K, _GROUP * _K)])
        return 0

    lax.fori_loop(0, _ROWS // _GROUP, group_body, 0)


@jax.jit
def _run(x0, x1, x2, kern_flat, op0):
    f32, i32 = jnp.float32, jnp.int32
    return pl.kernel(
        _sc_body,
        out_type=[
            jax.ShapeDtypeStruct((_B * _N * _K,), i32),
            jax.ShapeDtypeStruct((_B * _N * _K * _KS,), f32),
        ],
        mesh=plsc.VectorSubcoreMesh(core_axis_name="c", subcore_axis_name="s",
                                    num_cores=2, num_subcores=16),
        compiler_params=pltpu.CompilerParams(needs_layout_passes=False),
        scratch_types=[
            pltpu.VMEM((_N,), f32),
            pltpu.VMEM((_N,), f32),
            pltpu.VMEM((_N,), f32),
            pltpu.VMEM((_N,), f32),
            pltpu.VMEM((_N,), f32),
            pltpu.VMEM((_N,), f32),
            pltpu.VMEM((_N,), f32),
            pltpu.VMEM((_N,), f32),
            pltpu.VMEM((_K * _L,), f32),
            pltpu.VMEM((_GROUP * _K,), i32),
            pltpu.VMEM((_GROUP * _K * _KS,), f32),
            pltpu.VMEM((_F * _L,), f32),
            pltpu.VMEM((_L,), f32),
        ],
    )(x0, x1, x2, kern_flat, op0)


def kernel(x, kernals, kernals_padding, one_padding):
    x0 = x[:, 0, :].reshape(-1)
    x1 = x[:, 1, :].reshape(-1)
    x2 = x[:, 2, :].reshape(-1)
    kern_flat = jnp.concatenate([kernals_padding, kernals], axis=1).reshape(-1)
    op0 = one_padding[0]
    sidx, adjf = _run(x0, x1, x2, kern_flat, op0)
    return (sidx, adjf.reshape(_B * _N, _K, _KS))


# speedup vs baseline: 1.0589x; 1.0589x over previous
import functools

import jax
import jax.numpy as jnp
from jax import lax
from jax.experimental import pallas as pl
from jax.experimental.pallas import tpu as pltpu
from jax.experimental.pallas import tpu_sc as plsc

_B, _F, _N = 8, 3, 2048
_K = 32
_KS = 16
_L = 16
_NW = 32
_ROWS = (_B * _N) // _NW
_WPB = _NW // _B
_CHUNKS = _N // _L
_CAPV = 14
_CAP = _CAPV * _L
_PRUNE_AT = 160
_GROUP = 16
_NEG = float("-inf")


def _sort_desc(k, v):
    return plsc.sort_key_val(k, v, descending=True)


_GDN = lax.GatherDimensionNumbers(offset_dims=(), collapsed_slice_dims=(0,),
                                  start_index_map=(0,))


def _reg_gather(v, idx):
    return lax.gather(v, idx.reshape(_L, 1), _GDN, (1,),
                      mode=lax.GatherScatterMode.PROMISE_IN_BOUNDS)


def _bf16_round(v):
    u = plsc.bitcast(v, jnp.uint32)
    r = u + jnp.uint32(0x7FFF) + ((u >> jnp.uint32(16)) & jnp.uint32(1))
    r = r & jnp.uint32(0xFFFF0000)
    return plsc.bitcast(r, jnp.float32)


def _rev(a):
    return lax.rev(a, dimensions=(0,))


def _merge16(ak, av, bk, bv):
    brk, brv = _rev(bk), _rev(bv)
    m = ak >= brk
    hk = jnp.where(m, ak, brk)
    hv = jnp.where(m, av, brv)
    lk = jnp.where(m, brk, ak)
    lv = jnp.where(m, brv, av)
    hk, hv = _sort_desc(hk, hv)
    lk, lv = _sort_desc(lk, lv)
    return hk, hv, lk, lv


def _merge32_top32(a0k, a0v, a1k, a1v, b0k, b0v, b1k, b1v):
    rb0k, rb0v = _rev(b0k), _rev(b0v)
    rb1k, rb1v = _rev(b1k), _rev(b1v)
    m0 = a0k >= rb1k
    h0k = jnp.where(m0, a0k, rb1k)
    h0v = jnp.where(m0, a0v, rb1v)
    m1 = a1k >= rb0k
    h1k = jnp.where(m1, a1k, rb0k)
    h1v = jnp.where(m1, a1v, rb0v)
    m = h0k >= h1k
    pk = jnp.where(m, h0k, h1k)
    pv = jnp.where(m, h0v, h1v)
    qk = jnp.where(m, h1k, h0k)
    qv = jnp.where(m, h1v, h0v)
    pk, pv = _sort_desc(pk, pv)
    qk, qv = _sort_desc(qk, qv)
    return pk, pv, qk, qv


def _sc_body(x0_hbm, x1_hbm, x2_hbm, kern_hbm, op0_hbm, sidx_hbm, adj_hbm,
             x0_v, x1_v, x2_v, xb0_v, xb1_v, xb2_v, xx_v, keybuf, idxbuf,
             wbuf, sidx_stage, adj_stage, kern_v, op0_v):
    cid = lax.axis_index("c")
    sid = lax.axis_index("s")
    w = sid * 2 + cid
    b = w // _WPB
    rbase = (w % _WPB) * _ROWS

    pltpu.sync_copy(x0_hbm.at[pl.ds(b * _N, _N)], x0_v)
    pltpu.sync_copy(x1_hbm.at[pl.ds(b * _N, _N)], x1_v)
    pltpu.sync_copy(x2_hbm.at[pl.ds(b * _N, _N)], x2_v)
    pltpu.sync_copy(kern_hbm, kern_v)
    pltpu.sync_copy(op0_hbm, op0_v)

    def xx_body(i, _):
        a0 = x0_v[pl.ds(i * _L, _L)]
        a1 = x1_v[pl.ds(i * _L, _L)]
        a2 = x2_v[pl.ds(i * _L, _L)]
        xx_v[pl.ds(i * _L, _L)] = a0 * a0 + a1 * a1 + a2 * a2
        xb0_v[pl.ds(i * _L, _L)] = _bf16_round(a0)
        xb1_v[pl.ds(i * _L, _L)] = _bf16_round(a1)
        xb2_v[pl.ds(i * _L, _L)] = _bf16_round(a2)
        return 0
    lax.fori_loop(0, _CHUNKS, xx_body, 0)

    kv0 = _bf16_round(kern_v[pl.ds(0, _L)])
    kv1 = _bf16_round(kern_v[pl.ds(_L, _L)])
    kv2 = _bf16_round(kern_v[pl.ds(2 * _L, _L)])
    opv = op0_v[...]
    iota = lax.iota(jnp.int32, _L)
    bfly = [iota ^ 8, iota ^ 4, iota ^ 2, iota ^ 1]
    ninf = jnp.full((_L,), _NEG, jnp.float32)
    zero_i = jnp.zeros((_L,), jnp.int32)
    boff = b * _N

    def select_top32():
        runs = []
        for i in range(_CAPV):
            ki = keybuf[pl.ds(i * _L, _L)]
            vi = idxbuf[pl.ds(i * _L, _L)]
            runs.append(_sort_desc(ki, vi))
        runs32 = []
        for i in range(0, _CAPV, 2):
            runs32.append(_merge16(runs[i][0], runs[i][1],
                                   runs[i + 1][0], runs[i + 1][1]))
        cur = runs32[0]
        for i in range(1, len(runs32)):
            cur = _merge32_top32(*cur, *runs32[i])
        return cur

    def process_row(r, rr):
        row = rbase + r
        qi = jnp.full((_L,), row, jnp.int32)
        q0 = plsc.load_gather(xb0_v, [qi])
        q1 = plsc.load_gather(xb1_v, [qi])
        q2 = plsc.load_gather(xb2_v, [qi])

        def score_chunk(c):
            a0 = xb0_v[pl.ds(c * _L, _L)]
            a1 = xb1_v[pl.ds(c * _L, _L)]
            a2 = xb2_v[pl.ds(c * _L, _L)]
            xxv = xx_v[pl.ds(c * _L, _L)]
            dot = q0 * a0 + q1 * a1 + q2 * a2
            return dot + dot - xxv

        s0 = score_chunk(0)
        s1 = score_chunk(1)
        k0, v0 = _sort_desc(s0, iota)
        k1, v1 = _sort_desc(s1, iota + _L)
        hk, hv, lk, lv = _merge16(k0, v0, k1, v1)
        keybuf[pl.ds(0, _L)] = hk
        idxbuf[pl.ds(0, _L)] = hv
        keybuf[pl.ds(_L, _L)] = lk
        idxbuf[pl.ds(_L, _L)] = lv
        for i in range(2, _CAPV):
            keybuf[pl.ds(i * _L, _L)] = ninf
            idxbuf[pl.ds(i * _L, _L)] = zero_i
        t0 = jnp.min(lk)

        def prune(ct):
            pk, pv, qk, qv = select_top32()
            keybuf[pl.ds(0, _L)] = pk
            idxbuf[pl.ds(0, _L)] = pv
            keybuf[pl.ds(_L, _L)] = qk
            idxbuf[pl.ds(_L, _L)] = qv
            for i in range(2, _CAPV):
                keybuf[pl.ds(i * _L, _L)] = ninf
                idxbuf[pl.ds(i * _L, _L)] = zero_i
            return jnp.int32(_K), jnp.min(qk)

        def append_chunk(c, cnt, t):
            s = score_chunk(c)
            m = s > t
            plsc.store_compressed(keybuf.at[pl.ds(cnt, _L)], s, mask=m)
            plsc.store_compressed(idxbuf.at[pl.ds(cnt, _L)], iota + c * _L,
                                  mask=m)
            mi = m.astype(jnp.int32)
            for p in bfly:
                mi = mi + _reg_gather(mi, p)
            return cnt + mi[0]

        def blk_body(blk, carry):
            cnt, t = carry
            base = 2 + blk * 4
            for u in range(4):
                cnt = append_chunk(base + u, cnt, t)
            return lax.cond(cnt >= _PRUNE_AT, prune, lambda ct: ct, (cnt, t))

        cnt, t = lax.fori_loop(0, (_CHUNKS - 4) // 4, blk_body,
                               (jnp.int32(_K), t0))
        for c in (_CHUNKS - 2, _CHUNKS - 1):
            cnt = append_chunk(c, cnt, t)

        pk, pv, qk, qv = select_top32()
        i0, i1 = pv, qv

        sidx_stage[pl.ds(rr * _K, _L)] = i0 + boff
        sidx_stage[pl.ds(rr * _K + _L, _L)] = i1 + boff

        g0a = plsc.load_gather(x0_v, [i0])
        g1a = plsc.load_gather(x1_v, [i0])
        g2a = plsc.load_gather(x2_v, [i0])
        g0b = plsc.load_gather(x0_v, [i1])
        g1b = plsc.load_gather(x1_v, [i1])
        g2b = plsc.load_gather(x2_v, [i1])
        c0 = _reg_gather(g0a, zero_i)
        c1 = _reg_gather(g1a, zero_i)
        c2 = _reg_gather(g2a, zero_i)
        db0a = _bf16_round(g0a - c0)
        db0b = _bf16_round(g0b - c0)
        db1a = _bf16_round(g1a - c1)
        db1b = _bf16_round(g1b - c1)
        db2a = _bf16_round(g2a - c2)
        db2b = _bf16_round(g2b - c2)

        ssum = jnp.zeros((_L,), jnp.float32)
        for j in range(_K):
            jv = jnp.full((_L,), j % _L, jnp.int32)
            if j < _L:
                d0 = _reg_gather(db0a, jv)
                d1 = _reg_gather(db1a, jv)
                d2 = _reg_gather(db2a, jv)
            else:
                d0 = _reg_gather(db0b, jv)
                d1 = _reg_gather(db1b, jv)
                d2 = _reg_gather(db2b, jv)
            wj = d0 * kv0 + d1 * kv1 + d2 * kv2
            if j == 0:
                wj = wj + opv
            wj = jnp.maximum(wj, 0.0)
            ssum = ssum + wj
            wbuf[pl.ds(j * _L, _L)] = wj

        r1 = 1.0 / (ssum + 1e-6)
        s2 = jnp.zeros((_L,), jnp.float32)
        for j in range(_K):
            a = wbuf[pl.ds(j * _L, _L)] * r1
            a = a * a
            s2 = s2 + a
            wbuf[pl.ds(j * _L, _L)] = a

        r2 = 1.0 / (s2 + 1e-6)
        for j in range(_K):
            v = wbuf[pl.ds(j * _L, _L)] * r2
            v = jnp.where(v > 0.1, v, 0.0)
            adj_stage[pl.ds((rr * _K + j) * _KS, _KS)] = v
        return 0

    def group_body(g, _):
        def row_body(rr, _):
            return process_row(g * _GROUP + rr, rr)
        lax.fori_loop(0, _GROUP, row_body, 0)
        base = w * _ROWS + g * _GROUP
        pltpu.sync_copy(adj_stage,
                        adj_hbm.at[pl.ds(base * _K * _KS, _GROUP * _K * _KS)])
        pltpu.sync_copy(sidx_stage,
                        sidx_hbm.at[pl.ds(base * _K, _GROUP * _K)])
        return 0

    lax.fori_loop(0, _ROWS // _GROUP, group_body, 0)


@jax.jit
def _run(x0, x1, x2, kern_flat, op0):
    f32, i32 = jnp.float32, jnp.int32
    return pl.kernel(
        _sc_body,
        out_type=[
            jax.ShapeDtypeStruct((_B * _N * _K,), i32),
            jax.ShapeDtypeStruct((_B * _N * _K * _KS,), f32),
        ],
        mesh=plsc.VectorSubcoreMesh(core_axis_name="c", subcore_axis_name="s",
                                    num_cores=2, num_subcores=16),
        compiler_params=pltpu.CompilerParams(needs_layout_passes=False),
        scratch_types=[
            pltpu.VMEM((_N,), f32),
            pltpu.VMEM((_N,), f32),
            pltpu.VMEM((_N,), f32),
            pltpu.VMEM((_N,), f32),
            pltpu.VMEM((_N,), f32),
            pltpu.VMEM((_N,), f32),
            pltpu.VMEM((_N,), f32),
            pltpu.VMEM((_CAP,), f32),
            pltpu.VMEM((_CAP,), i32),
            pltpu.VMEM((_K * _L,), f32),
            pltpu.VMEM((_GROUP * _K,), i32),
            pltpu.VMEM((_GROUP * _K * _KS,), f32),
            pltpu.VMEM((_F * _L,), f32),
            pltpu.VMEM((_L,), f32),
        ],
    )(x0, x1, x2, kern_flat, op0)


def kernel(x, kernals, kernals_padding, one_padding):
    x0 = x[:, 0, :].reshape(-1)
    x1 = x[:, 1, :].reshape(-1)
    x2 = x[:, 2, :].reshape(-1)
    kern_flat = jnp.concatenate([kernals_padding, kernals], axis=1).reshape(-1)
    op0 = one_padding[0]
    sidx, adjf = _run(x0, x1, x2, kern_flat, op0)
    return (sidx, adjf.reshape(_B * _N, _K, _KS))


# speedup vs baseline: 1.1493x; 1.0854x over previous
import functools

import jax
import jax.numpy as jnp
from jax import lax
from jax.experimental import pallas as pl
from jax.experimental.pallas import tpu as pltpu
from jax.experimental.pallas import tpu_sc as plsc

_B, _F, _N = 8, 3, 2048
_K = 32
_KS = 16
_L = 16
_NW = 32
_ROWS = (_B * _N) // _NW
_WPB = _NW // _B
_CHUNKS = _N // _L
_CAPV = 14
_CAP = _CAPV * _L
_PRUNE_AT = 160
_GROUP = 16
_NEG = float("-inf")


def _sort_desc(k, v):
    return plsc.sort_key_val(k, v, descending=True)


_GDN = lax.GatherDimensionNumbers(offset_dims=(), collapsed_slice_dims=(0,),
                                  start_index_map=(0,))


def _reg_gather(v, idx):
    return lax.gather(v, idx.reshape(_L, 1), _GDN, (1,),
                      mode=lax.GatherScatterMode.PROMISE_IN_BOUNDS)


def _bf16_round(v):
    u = plsc.bitcast(v, jnp.uint32)
    r = u + jnp.uint32(0x7FFF) + ((u >> jnp.uint32(16)) & jnp.uint32(1))
    r = r & jnp.uint32(0xFFFF0000)
    return plsc.bitcast(r, jnp.float32)


def _rev(a):
    return lax.rev(a, dimensions=(0,))


def _merge16(ak, av, bk, bv):
    brk, brv = _rev(bk), _rev(bv)
    m = ak >= brk
    hk = jnp.where(m, ak, brk)
    hv = jnp.where(m, av, brv)
    lk = jnp.where(m, brk, ak)
    lv = jnp.where(m, brv, av)
    hk, hv = _sort_desc(hk, hv)
    lk, lv = _sort_desc(lk, lv)
    return hk, hv, lk, lv


def _merge32_top32(a0k, a0v, a1k, a1v, b0k, b0v, b1k, b1v):
    rb0k, rb0v = _rev(b0k), _rev(b0v)
    rb1k, rb1v = _rev(b1k), _rev(b1v)
    m0 = a0k >= rb1k
    h0k = jnp.where(m0, a0k, rb1k)
    h0v = jnp.where(m0, a0v, rb1v)
    m1 = a1k >= rb0k
    h1k = jnp.where(m1, a1k, rb0k)
    h1v = jnp.where(m1, a1v, rb0v)
    m = h0k >= h1k
    pk = jnp.where(m, h0k, h1k)
    pv = jnp.where(m, h0v, h1v)
    qk = jnp.where(m, h1k, h0k)
    qv = jnp.where(m, h1v, h0v)
    pk, pv = _sort_desc(pk, pv)
    qk, qv = _sort_desc(qk, qv)
    return pk, pv, qk, qv


def _sc_body(x0_hbm, x1_hbm, x2_hbm, kern_hbm, op0_hbm, sidx_hbm, adj_hbm,
             x0_v, x1_v, x2_v, xb0_v, xb1_v, xb2_v, xx_v, keybuf, idxbuf,
             wbuf, sidx_stage, adj_stage, kern_v, op0_v):
    cid = lax.axis_index("c")
    sid = lax.axis_index("s")
    w = sid * 2 + cid
    b = w // _WPB
    rbase = (w % _WPB) * _ROWS

    pltpu.sync_copy(x0_hbm.at[pl.ds(b * _N, _N)], x0_v)
    pltpu.sync_copy(x1_hbm.at[pl.ds(b * _N, _N)], x1_v)
    pltpu.sync_copy(x2_hbm.at[pl.ds(b * _N, _N)], x2_v)
    pltpu.sync_copy(kern_hbm, kern_v)
    pltpu.sync_copy(op0_hbm, op0_v)

    def xx_body(i, _):
        a0 = x0_v[pl.ds(i * _L, _L)]
        a1 = x1_v[pl.ds(i * _L, _L)]
        a2 = x2_v[pl.ds(i * _L, _L)]
        xx_v[pl.ds(i * _L, _L)] = a0 * a0 + a1 * a1 + a2 * a2
        xb0_v[pl.ds(i * _L, _L)] = _bf16_round(a0)
        xb1_v[pl.ds(i * _L, _L)] = _bf16_round(a1)
        xb2_v[pl.ds(i * _L, _L)] = _bf16_round(a2)
        return 0
    lax.fori_loop(0, _CHUNKS, xx_body, 0)

    kv0 = _bf16_round(kern_v[pl.ds(0, _L)])
    kv1 = _bf16_round(kern_v[pl.ds(_L, _L)])
    kv2 = _bf16_round(kern_v[pl.ds(2 * _L, _L)])
    opv = op0_v[...]
    iota = lax.iota(jnp.int32, _L)
    ninf = jnp.full((_L,), _NEG, jnp.float32)
    zero_i = jnp.zeros((_L,), jnp.int32)
    boff = b * _N

    def select_top32():
        runs = []
        for i in range(_CAPV):
            ki = keybuf[pl.ds(i * _L, _L)]
            vi = idxbuf[pl.ds(i * _L, _L)]
            runs.append(_sort_desc(ki, vi))
        runs32 = []
        for i in range(0, _CAPV, 2):
            runs32.append(_merge16(runs[i][0], runs[i][1],
                                   runs[i + 1][0], runs[i + 1][1]))
        cur = runs32[0]
        for i in range(1, len(runs32)):
            cur = _merge32_top32(*cur, *runs32[i])
        return cur

    def process_row(r, rr):
        row = rbase + r
        qi = jnp.full((_L,), row, jnp.int32)
        q0 = plsc.load_gather(xb0_v, [qi])
        q1 = plsc.load_gather(xb1_v, [qi])
        q2 = plsc.load_gather(xb2_v, [qi])

        def score_chunk(c):
            a0 = xb0_v[pl.ds(c * _L, _L)]
            a1 = xb1_v[pl.ds(c * _L, _L)]
            a2 = xb2_v[pl.ds(c * _L, _L)]
            xxv = xx_v[pl.ds(c * _L, _L)]
            dot = q0 * a0 + q1 * a1 + q2 * a2
            return dot + dot - xxv

        s0 = score_chunk(0)
        s1 = score_chunk(1)
        k0, v0 = _sort_desc(s0, iota)
        k1, v1 = _sort_desc(s1, iota + _L)
        hk, hv, lk, lv = _merge16(k0, v0, k1, v1)
        keybuf[pl.ds(0, _L)] = hk
        idxbuf[pl.ds(0, _L)] = hv
        keybuf[pl.ds(_L, _L)] = lk
        idxbuf[pl.ds(_L, _L)] = lv
        for i in range(2, _CAPV):
            keybuf[pl.ds(i * _L, _L)] = ninf
            idxbuf[pl.ds(i * _L, _L)] = zero_i
        t0 = jnp.min(lk)

        def prune(ct):
            pk, pv, qk, qv = select_top32()
            keybuf[pl.ds(0, _L)] = pk
            idxbuf[pl.ds(0, _L)] = pv
            keybuf[pl.ds(_L, _L)] = qk
            idxbuf[pl.ds(_L, _L)] = qv
            for i in range(2, _CAPV):
                keybuf[pl.ds(i * _L, _L)] = ninf
                idxbuf[pl.ds(i * _L, _L)] = zero_i
            return jnp.int32(_K), jnp.min(qk)

        def append_chunk(c, cnt, t):
            s = score_chunk(c)
            m = s > t
            plsc.store_compressed(keybuf.at[pl.ds(cnt, _L)], s, mask=m)
            plsc.store_compressed(idxbuf.at[pl.ds(cnt, _L)], iota + c * _L,
                                  mask=m)
            return cnt + plsc.all_reduce_population_count(m)[0]

        def blk_body(blk, carry):
            cnt, t = carry
            base = 2 + blk * 4
            for u in range(4):
                cnt = append_chunk(base + u, cnt, t)
            return lax.cond(cnt >= _PRUNE_AT, prune, lambda ct: ct, (cnt, t))

        cnt, t = lax.fori_loop(0, (_CHUNKS - 4) // 4, blk_body,
                               (jnp.int32(_K), t0))
        for c in (_CHUNKS - 2, _CHUNKS - 1):
            cnt = append_chunk(c, cnt, t)

        pk, pv, qk, qv = select_top32()
        i0, i1 = pv, qv

        sidx_stage[pl.ds(rr * _K, _L)] = i0 + boff
        sidx_stage[pl.ds(rr * _K + _L, _L)] = i1 + boff

        g0a = plsc.load_gather(x0_v, [i0])
        g1a = plsc.load_gather(x1_v, [i0])
        g2a = plsc.load_gather(x2_v, [i0])
        g0b = plsc.load_gather(x0_v, [i1])
        g1b = plsc.load_gather(x1_v, [i1])
        g2b = plsc.load_gather(x2_v, [i1])
        c0 = _reg_gather(g0a, zero_i)
        c1 = _reg_gather(g1a, zero_i)
        c2 = _reg_gather(g2a, zero_i)
        db0a = _bf16_round(g0a - c0)
        db0b = _bf16_round(g0b - c0)
        db1a = _bf16_round(g1a - c1)
        db1b = _bf16_round(g1b - c1)
        db2a = _bf16_round(g2a - c2)
        db2b = _bf16_round(g2b - c2)

        ssum = jnp.zeros((_L,), jnp.float32)
        for j in range(_K):
            jv = jnp.full((_L,), j % _L, jnp.int32)
            if j < _L:
                d0 = _reg_gather(db0a, jv)
                d1 = _reg_gather(db1a, jv)
                d2 = _reg_gather(db2a, jv)
            else:
                d0 = _reg_gather(db0b, jv)
                d1 = _reg_gather(db1b, jv)
                d2 = _reg_gather(db2b, jv)
            wj = d0 * kv0 + d1 * kv1 + d2 * kv2
            if j == 0:
                wj = wj + opv
            wj = jnp.maximum(wj, 0.0)
            ssum = ssum + wj
            wbuf[pl.ds(j * _L, _L)] = wj

        r1 = 1.0 / (ssum + 1e-6)
        s2 = jnp.zeros((_L,), jnp.float32)
        for j in range(_K):
            a = wbuf[pl.ds(j * _L, _L)] * r1
            a = a * a
            s2 = s2 + a
            wbuf[pl.ds(j * _L, _L)] = a

        r2 = 1.0 / (s2 + 1e-6)
        for j in range(_K):
            v = wbuf[pl.ds(j * _L, _L)] * r2
            v = jnp.where(v > 0.1, v, 0.0)
            adj_stage[pl.ds((rr * _K + j) * _KS, _KS)] = v
        return 0

    def group_body(g, _):
        def row_body(rr, _):
            return process_row(g * _GROUP + rr, rr)
        lax.fori_loop(0, _GROUP, row_body, 0)
        base = w * _ROWS + g * _GROUP
        pltpu.sync_copy(adj_stage,
                        adj_hbm.at[pl.ds(base * _K * _KS, _GROUP * _K * _KS)])
        pltpu.sync_copy(sidx_stage,
                        sidx_hbm.at[pl.ds(base * _K, _GROUP * _K)])
        return 0

    lax.fori_loop(0, _ROWS // _GROUP, group_body, 0)


@jax.jit
def _run(x0, x1, x2, kern_flat, op0):
    f32, i32 = jnp.float32, jnp.int32
    return pl.kernel(
        _sc_body,
        out_type=[
            jax.ShapeDtypeStruct((_B * _N * _K,), i32),
            jax.ShapeDtypeStruct((_B * _N * _K * _KS,), f32),
        ],
        mesh=plsc.VectorSubcoreMesh(core_axis_name="c", subcore_axis_name="s",
                                    num_cores=2, num_subcores=16),
        compiler_params=pltpu.CompilerParams(needs_layout_passes=False),
        scratch_types=[
            pltpu.VMEM((_N,), f32),
            pltpu.VMEM((_N,), f32),
            pltpu.VMEM((_N,), f32),
            pltpu.VMEM((_N,), f32),
            pltpu.VMEM((_N,), f32),
            pltpu.VMEM((_N,), f32),
            pltpu.VMEM((_N,), f32),
            pltpu.VMEM((_CAP,), f32),
            pltpu.VMEM((_CAP,), i32),
            pltpu.VMEM((_K * _L,), f32),
            pltpu.VMEM((_GROUP * _K,), i32),
            pltpu.VMEM((_GROUP * _K * _KS,), f32),
            pltpu.VMEM((_F * _L,), f32),
            pltpu.VMEM((_L,), f32),
        ],
    )(x0, x1, x2, kern_flat, op0)


def kernel(x, kernals, kernals_padding, one_padding):
    x0 = x[:, 0, :].reshape(-1)
    x1 = x[:, 1, :].reshape(-1)
    x2 = x[:, 2, :].reshape(-1)
    kern_flat = jnp.concatenate([kernals_padding, kernals], axis=1).reshape(-1)
    op0 = one_padding[0]
    sidx, adjf = _run(x0, x1, x2, kern_flat, op0)
    return (sidx, adjf.reshape(_B * _N, _K, _KS))
